# baseline (speedup 1.0000x reference)
import functools

import jax
import jax.numpy as jnp
from jax import lax
from jax.experimental import pallas as pl
from jax.experimental.pallas import tpu as pltpu
from jax.experimental.pallas import tpu_sc as plsc

N = 1048576
NUM_CORES = 2
NUM_SUBCORES = 16
NW = NUM_CORES * NUM_SUBCORES
ROWS_PER_W = N // NW
CHUNK_ROWS = 8192
NUM_CHUNKS = ROWS_PER_W // CHUNK_ROWS
LANES = 16


def _body(pos_hbm, vel_hbm, diam_hbm,
          dpos_hbm, dvel_hbm, ddiam_hbm,
          pos_v, vel_v, diam_v, dpos_v, dvel_v, ddiam_v):
    wid = lax.axis_index("s") * NUM_CORES + lax.axis_index("c")

    lane = lax.iota(jnp.int32, LANES)
    ones_i = jnp.ones((LANES,), jnp.int32)
    half = lax.shift_right_logical(lane, ones_i)
    yoff = lax.shift_left(half, ones_i) + ones_i
    pat = (lane & ones_i).astype(jnp.float32) * jnp.full(
        (LANES,), -20.0, jnp.float32)
    zeros = jnp.zeros((LANES,), jnp.float32)
    halves = jnp.full((LANES,), 0.5, jnp.float32)

    def zero_body(k, carry):
        ddiam_v[pl.ds(k * LANES, LANES)] = zeros
        return carry
    lax.fori_loop(0, CHUNK_ROWS // LANES, zero_body, 0)

    for c in range(NUM_CHUNKS):
        row0 = wid * ROWS_PER_W + c * CHUNK_ROWS
        e0 = 2 * row0
        pltpu.sync_copy(pos_hbm.at[pl.ds(e0, 2 * CHUNK_ROWS)], pos_v)
        pltpu.sync_copy(vel_hbm.at[pl.ds(e0, 2 * CHUNK_ROWS)], vel_v)
        pltpu.sync_copy(diam_hbm.at[pl.ds(row0, CHUNK_ROWS)], diam_v)

        def body(k, carry):
            eb = k * LANES
            rb = k * (LANES // 2)
            v = vel_v[pl.ds(eb, LANES)]
            y = plsc.load_gather(pos_v, [jnp.full((LANES,), eb, jnp.int32) + yoff])
            d = plsc.load_gather(diam_v, [jnp.full((LANES,), rb, jnp.int32) + half])
            m = y <= halves * d
            dpos_v[pl.ds(eb, LANES)] = jnp.where(m, zeros, v)
            dvel_v[pl.ds(eb, LANES)] = jnp.where(m, zeros, pat)
            return carry
        lax.fori_loop(0, (2 * CHUNK_ROWS) // LANES, body, 0)

        pltpu.sync_copy(dpos_v, dpos_hbm.at[pl.ds(e0, 2 * CHUNK_ROWS)])
        pltpu.sync_copy(dvel_v, dvel_hbm.at[pl.ds(e0, 2 * CHUNK_ROWS)])
        pltpu.sync_copy(ddiam_v, ddiam_hbm.at[pl.ds(row0, CHUNK_ROWS)])


_sc_call = functools.partial(
    pl.kernel,
    out_type=(
        jax.ShapeDtypeStruct((2 * N,), jnp.float32),
        jax.ShapeDtypeStruct((2 * N,), jnp.float32),
        jax.ShapeDtypeStruct((N,), jnp.float32),
    ),
    mesh=plsc.VectorSubcoreMesh(core_axis_name="c", subcore_axis_name="s"),
    compiler_params=pltpu.CompilerParams(needs_layout_passes=False),
    scratch_types=[
        pltpu.VMEM((2 * CHUNK_ROWS,), jnp.float32),
        pltpu.VMEM((2 * CHUNK_ROWS,), jnp.float32),
        pltpu.VMEM((CHUNK_ROWS,), jnp.float32),
        pltpu.VMEM((2 * CHUNK_ROWS,), jnp.float32),
        pltpu.VMEM((2 * CHUNK_ROWS,), jnp.float32),
        pltpu.VMEM((CHUNK_ROWS,), jnp.float32),
    ],
)(_body)


@jax.jit
def kernel(t, pos, vel, diameter):
    del t
    dpos_f, dvel_f, ddiam = _sc_call(
        pos.reshape(-1), vel.reshape(-1), diameter)
    return (dpos_f.reshape(N, 2), dvel_f.reshape(N, 2), ddiam)
